# baseline (speedup 1.0000x reference)
import functools

import jax
import jax.numpy as jnp
from jax import lax
from jax.experimental import pallas as pl
from jax.experimental.pallas import tpu as pltpu
from jax.experimental.pallas import tpu_sc as plsc

_B = 16384
_EMB = 128
_NG = 100

_NC, _NS = 2, 16
_NW = _NC * _NS
_BPW = _B // _NW
_CHUNK = 128
_NCHUNK = _BPW // _CHUNK


def _sc_gather(table, idx):
  mesh = plsc.VectorSubcoreMesh(core_axis_name="c", subcore_axis_name="s")

  @functools.partial(
      pl.kernel,
      mesh=mesh,
      out_type=jax.ShapeDtypeStruct((_B, _EMB), jnp.float32),
      scratch_types=[
          pltpu.VMEM((_NCHUNK, _CHUNK), jnp.int32),
          pltpu.VMEM((_BPW, _EMB), jnp.float32),
          pltpu.SemaphoreType.DMA,
      ],
  )
  def body(table_hbm, idx_hbm, out_hbm, idx_v, rows_v, sem):
    wid = lax.axis_index("s") * _NC + lax.axis_index("c")
    base = wid * _BPW
    for j in range(_NCHUNK):
      pltpu.sync_copy(idx_hbm.at[pl.ds(base + j * _CHUNK, _CHUNK)],
                      idx_v.at[j])
    copies = [
        pltpu.async_copy(table_hbm.at[idx_v.at[j]],
                         rows_v.at[pl.ds(j * _CHUNK, _CHUNK)], sem)
        for j in range(_NCHUNK)
    ]
    for c in copies:
      c.wait()
    pltpu.sync_copy(rows_v, out_hbm.at[pl.ds(base, _BPW)])

  return body(table, idx)


_BM = 1024


def _tc_body(x_ref, rate_ref, w_ref, out_ref):
  xb = x_ref[...].astype(jnp.bfloat16)
  acc = lax.dot_general(xb, w_ref[...],
                        (((1,), (1,)), ((), ())),
                        preferred_element_type=jnp.float32)
  genre = acc[:, :_EMB] / acc[:, _EMB:_EMB + 1]
  out_ref[:, :_EMB] = rate_ref[...]
  out_ref[:, _EMB:] = genre


def _tc_fuse(x, rate_emb, w_aug):
  grid = (_B // _BM,)
  return pl.pallas_call(
      _tc_body,
      grid=grid,
      in_specs=[
          pl.BlockSpec((_BM, 1 + _NG), lambda i: (i, 0)),
          pl.BlockSpec((_BM, _EMB), lambda i: (i, 0)),
          pl.BlockSpec((_EMB + 1, 1 + _NG), lambda i: (0, 0)),
      ],
      out_specs=pl.BlockSpec((_BM, 2 * _EMB), lambda i: (i, 0)),
      out_shape=jax.ShapeDtypeStruct((_B, 2 * _EMB), jnp.float32),
      compiler_params=pltpu.CompilerParams(
          dimension_semantics=("parallel",),
      ),
  )(x, rate_emb, w_aug)


def kernel(x, embedding_rate, genre_weight):
  rate_idx = x[:, 0]
  w = jnp.pad(genre_weight, ((0, 1), (1, 0)))
  w_aug = (w.at[_EMB, 1:].set(1.0)).astype(jnp.bfloat16)
  rate_emb = _sc_gather(embedding_rate, rate_idx)
  return _tc_fuse(x, rate_emb, w_aug)


# speedup vs baseline: 7.5775x; 7.5775x over previous
import functools

import jax
import jax.numpy as jnp
from jax import lax
from jax.experimental import pallas as pl
from jax.experimental.pallas import tpu as pltpu
from jax.experimental.pallas import tpu_sc as plsc

_B = 16384
_EMB = 128
_NG = 100

_NC, _NS = 2, 16
_NW = _NC * _NS
_BPW = _B // _NW
_CHUNK = 128
_NCHUNK = _BPW // _CHUNK


_TPAD = 1024
_TSHARD = _TPAD // _NS


def _sc_gather(table, idx):
  mesh = plsc.VectorSubcoreMesh(core_axis_name="c", subcore_axis_name="s")

  @functools.partial(
      pl.kernel,
      mesh=mesh,
      out_type=jax.ShapeDtypeStruct((_B, _EMB), jnp.float32),
      scratch_types=[
          pltpu.VMEM((_NCHUNK, _CHUNK), jnp.int32),
          pltpu.VMEM((_BPW, _EMB), jnp.float32),
          pltpu.VMEM_SHARED((_TPAD, _EMB), jnp.float32),
          pltpu.SemaphoreType.DMA,
      ],
  )
  def body(table_hbm, idx_hbm, out_hbm, idx_v, rows_v, tab_sp, sem):
    cid = lax.axis_index("c")
    sid = lax.axis_index("s")
    wid = sid * _NC + cid
    base = wid * _BPW
    stage = pltpu.async_copy(table_hbm.at[pl.ds(sid * _TSHARD, _TSHARD)],
                             tab_sp.at[pl.ds(sid * _TSHARD, _TSHARD)], sem)
    for j in range(_NCHUNK):
      pltpu.sync_copy(idx_hbm.at[pl.ds(base + j * _CHUNK, _CHUNK)],
                      idx_v.at[j])
    stage.wait()
    plsc.subcore_barrier()
    copies = [
        pltpu.async_copy(tab_sp.at[idx_v.at[j]],
                         rows_v.at[pl.ds(j * _CHUNK, _CHUNK)], sem)
        for j in range(_NCHUNK)
    ]
    for c in copies:
      c.wait()
    pltpu.sync_copy(rows_v, out_hbm.at[pl.ds(base, _BPW)])

  return body(table, idx)


_BM = 1024


def _tc_body(x_ref, rate_ref, w_ref, out_ref):
  xb = x_ref[...].astype(jnp.bfloat16)
  acc = lax.dot_general(xb, w_ref[...],
                        (((1,), (1,)), ((), ())),
                        preferred_element_type=jnp.float32)
  genre = acc[:, :_EMB] / acc[:, _EMB:_EMB + 1]
  out_ref[:, :_EMB] = rate_ref[...]
  out_ref[:, _EMB:] = genre


def _tc_fuse(x, rate_emb, w_aug):
  grid = (_B // _BM,)
  return pl.pallas_call(
      _tc_body,
      grid=grid,
      in_specs=[
          pl.BlockSpec((_BM, 1 + _NG), lambda i: (i, 0)),
          pl.BlockSpec((_BM, _EMB), lambda i: (i, 0)),
          pl.BlockSpec((_EMB + 1, 1 + _NG), lambda i: (0, 0)),
      ],
      out_specs=pl.BlockSpec((_BM, 2 * _EMB), lambda i: (i, 0)),
      out_shape=jax.ShapeDtypeStruct((_B, 2 * _EMB), jnp.float32),
      compiler_params=pltpu.CompilerParams(
          dimension_semantics=("parallel",),
      ),
  )(x, rate_emb, w_aug)


def kernel(x, embedding_rate, genre_weight):
  rate_idx = x[:, 0]
  w = jnp.pad(genre_weight, ((0, 1), (1, 0)))
  w_aug = (w.at[_EMB, 1:].set(1.0)).astype(jnp.bfloat16)
  table_pad = jnp.pad(embedding_rate, ((0, _TPAD - embedding_rate.shape[0]),
                                       (0, 0)))
  rate_emb = _sc_gather(table_pad, rate_idx)
  return _tc_fuse(x, rate_emb, w_aug)


# speedup vs baseline: 8.2849x; 1.0934x over previous
import functools

import jax
import jax.numpy as jnp
from jax import lax
from jax.experimental import pallas as pl
from jax.experimental.pallas import tpu as pltpu
from jax.experimental.pallas import tpu_sc as plsc

_B = 16384
_EMB = 128
_NG = 100

_NC, _NS = 2, 16
_NW = _NC * _NS
_BPW = _B // _NW
_CHUNK = 128
_NCHUNK = _BPW // _CHUNK


_TPAD = 1024
_TSHARD = _TPAD // _NS


def _sc_gather(table, idx):
  mesh = plsc.VectorSubcoreMesh(core_axis_name="c", subcore_axis_name="s")

  @functools.partial(
      pl.kernel,
      mesh=mesh,
      out_type=jax.ShapeDtypeStruct((_B, 2 * _EMB), jnp.float32),
      scratch_types=[
          pltpu.VMEM((_NCHUNK, _CHUNK), jnp.int32),
          pltpu.VMEM((_BPW, _EMB), jnp.float32),
          pltpu.VMEM_SHARED((_TPAD, _EMB), jnp.float32),
          pltpu.SemaphoreType.DMA,
      ],
  )
  def body(table_hbm, idx_hbm, out_hbm, idx_v, rows_v, tab_sp, sem):
    cid = lax.axis_index("c")
    sid = lax.axis_index("s")
    wid = sid * _NC + cid
    base = wid * _BPW
    stage = pltpu.async_copy(table_hbm.at[pl.ds(sid * _TSHARD, _TSHARD)],
                             tab_sp.at[pl.ds(sid * _TSHARD, _TSHARD)], sem)
    for j in range(_NCHUNK):
      pltpu.sync_copy(idx_hbm.at[pl.ds(base + j * _CHUNK, _CHUNK)],
                      idx_v.at[j])
    stage.wait()
    plsc.subcore_barrier()
    copies = [
        pltpu.async_copy(tab_sp.at[idx_v.at[j]],
                         rows_v.at[pl.ds(j * _CHUNK, _CHUNK)], sem)
        for j in range(_NCHUNK)
    ]
    for c in copies:
      c.wait()
    pltpu.sync_copy(rows_v, out_hbm.at[pl.ds(base, _BPW), pl.ds(0, _EMB)])

  return body(table, idx)


_BM = 1024


def _tc_body(x_ref, w_ref, buf_ref, out_ref):
  del buf_ref
  xb = x_ref[...].astype(jnp.bfloat16)
  acc = lax.dot_general(xb, w_ref[...],
                        (((1,), (1,)), ((), ())),
                        preferred_element_type=jnp.float32)
  out_ref[...] = acc[:, :_EMB] / acc[:, _EMB:_EMB + 1]


def _tc_fuse(x, buf, w_aug):
  grid = (_B // _BM,)
  return pl.pallas_call(
      _tc_body,
      grid=grid,
      in_specs=[
          pl.BlockSpec((_BM, 1 + _NG), lambda i: (i, 0)),
          pl.BlockSpec((_EMB + 1, 1 + _NG), lambda i: (0, 0)),
          pl.BlockSpec(memory_space=pltpu.MemorySpace.HBM),
      ],
      out_specs=pl.BlockSpec((_BM, _EMB), lambda i: (i, 1)),
      out_shape=jax.ShapeDtypeStruct((_B, 2 * _EMB), jnp.float32),
      input_output_aliases={2: 0},
      compiler_params=pltpu.CompilerParams(
          dimension_semantics=("parallel",),
      ),
  )(x, w_aug, buf)


def kernel(x, embedding_rate, genre_weight):
  rate_idx = x[:, 0]
  w = jnp.pad(genre_weight, ((0, 1), (1, 0)))
  w_aug = (w.at[_EMB, 1:].set(1.0)).astype(jnp.bfloat16)
  table_pad = jnp.pad(embedding_rate, ((0, _TPAD - embedding_rate.shape[0]),
                                       (0, 0)))
  buf = _sc_gather(table_pad, rate_idx)
  return _tc_fuse(x, buf, w_aug)


# speedup vs baseline: 8.5607x; 1.0333x over previous
import functools

import jax
import jax.numpy as jnp
from jax import lax
from jax.experimental import pallas as pl
from jax.experimental.pallas import tpu as pltpu
from jax.experimental.pallas import tpu_sc as plsc

_B = 16384
_EMB = 128
_NG = 100

_NC, _NS = 2, 16
_NW = _NC * _NS
_BPW = _B // _NW
_CHUNK = 128
_NCHUNK = _BPW // _CHUNK


_NR = 1000
_TSHARD = 64


def _sc_gather(table, idx):
  mesh = plsc.VectorSubcoreMesh(core_axis_name="c", subcore_axis_name="s")

  @functools.partial(
      pl.kernel,
      mesh=mesh,
      out_type=jax.ShapeDtypeStruct((_B, 2 * _EMB), jnp.float32),
      scratch_types=[
          pltpu.VMEM((_NCHUNK, _CHUNK), jnp.int32),
          pltpu.VMEM((_BPW, _EMB), jnp.float32),
          pltpu.VMEM_SHARED((_NR, _EMB), jnp.float32),
          pltpu.SemaphoreType.DMA,
      ],
  )
  def body(table_hbm, idx_hbm, out_hbm, idx_v, rows_v, tab_sp, sem):
    cid = lax.axis_index("c")
    sid = lax.axis_index("s")
    wid = sid * _NC + cid
    base = wid * _BPW
    srow = jnp.minimum(sid * _TSHARD, _NR - _TSHARD)
    stage = pltpu.async_copy(table_hbm.at[pl.ds(srow, _TSHARD)],
                             tab_sp.at[pl.ds(srow, _TSHARD)], sem)
    for j in range(_NCHUNK):
      pltpu.sync_copy(idx_hbm.at[pl.ds(base + j * _CHUNK, _CHUNK)],
                      idx_v.at[j])
    stage.wait()
    plsc.subcore_barrier()
    copies = [
        pltpu.async_copy(tab_sp.at[idx_v.at[j]],
                         rows_v.at[pl.ds(j * _CHUNK, _CHUNK)], sem)
        for j in range(_NCHUNK)
    ]
    for c in copies:
      c.wait()
    pltpu.sync_copy(rows_v, out_hbm.at[pl.ds(base, _BPW), pl.ds(0, _EMB)])

  return body(table, idx)


_BM = 1024


def _tc_body(x_ref, w_ref, buf_ref, out_ref):
  del buf_ref
  xb = x_ref[...]
  xbf = xb.astype(jnp.bfloat16)
  w101 = jnp.concatenate(
      [jnp.zeros((_EMB, 1), jnp.bfloat16), w_ref[...].astype(jnp.bfloat16)],
      axis=1)
  acc = lax.dot_general(xbf, w101,
                        (((1,), (1,)), ((), ())),
                        preferred_element_type=jnp.float32)
  xf = xb.astype(jnp.float32)
  cnt = (jnp.sum(xf, axis=1, keepdims=True) - xf[:, 0:1])
  out_ref[...] = acc / cnt


def _tc_fuse(x, buf, w_aug):
  grid = (_B // _BM,)
  return pl.pallas_call(
      _tc_body,
      grid=grid,
      in_specs=[
          pl.BlockSpec((_BM, 1 + _NG), lambda i: (i, 0)),
          pl.BlockSpec((_EMB, _NG), lambda i: (0, 0)),
          pl.BlockSpec(memory_space=pltpu.MemorySpace.HBM),
      ],
      out_specs=pl.BlockSpec((_BM, _EMB), lambda i: (i, 1)),
      out_shape=jax.ShapeDtypeStruct((_B, 2 * _EMB), jnp.float32),
      input_output_aliases={2: 0},
      compiler_params=pltpu.CompilerParams(
          dimension_semantics=("parallel",),
      ),
  )(x, w_aug, buf)


def kernel(x, embedding_rate, genre_weight):
  rate_idx = x[:, 0]
  buf = _sc_gather(embedding_rate, rate_idx)
  return _tc_fuse(x, buf, genre_weight)
